# baseline (speedup 1.0000x reference)
import functools

import jax
import jax.numpy as jnp
from jax import lax
from jax.experimental import pallas as pl
from jax.experimental.pallas import tpu as pltpu
from jax.experimental.pallas import tpu_sc as plsc

VOC = 100000
EMB = 128
B = 4096
C = 5
R = 10
NCR = C + R

NW = 32
BPW = B // NW
SUB = 16
NSUB = BPW // SUB
CC = SUB * C
RC = SUB * R
LN2 = 0.6931471805599453


def _log_f32(x):
    bits = lax.bitcast_convert_type(x, jnp.int32)
    e = ((bits >> 23) & 0xFF) - 127
    m = lax.bitcast_convert_type(
        (bits & 0x7FFFFF) | 0x3F800000, jnp.float32)
    s = (m - 1.0) / (m + 1.0)
    s2 = s * s
    p = s * (2.0 + s2 * (2.0 / 3.0 + s2 * (2.0 / 5.0 + s2 * (2.0 / 7.0
             + s2 * (2.0 / 9.0)))))
    return e.astype(jnp.float32) * LN2 + p


def _sc_loss(center, ctx_idx, rnd_idx, emb_table, lin_w):
    mesh = plsc.VectorSubcoreMesh(core_axis_name="c", subcore_axis_name="s")

    @functools.partial(
        pl.kernel,
        mesh=mesh,
        out_type=jax.ShapeDtypeStruct((2, 16), jnp.float32),
        scratch_types=[
            pltpu.VMEM((BPW,), jnp.int32),
            pltpu.VMEM((BPW * C,), jnp.int32),
            pltpu.VMEM((BPW * R,), jnp.int32),
            pltpu.VMEM((BPW, EMB), jnp.float32),
            pltpu.VMEM((CC, EMB), jnp.float32),
            pltpu.VMEM((CC, EMB), jnp.float32),
            pltpu.VMEM((RC, EMB), jnp.float32),
            pltpu.VMEM((RC, EMB), jnp.float32),
            pltpu.VMEM((16,), jnp.float32),
            pltpu.VMEM((16, 16), jnp.float32),
            pltpu.VMEM_SHARED((16, 16), jnp.float32),
            pltpu.SemaphoreType.DMA,
            pltpu.SemaphoreType.DMA,
            pltpu.SemaphoreType.DMA,
        ],
    )
    def k(center_h, ctx_h, rnd_h, emb_h, lin_h, out_h,
          cidx_v, xidx_v, ridx_v, erows_v, xrows0_v, xrows1_v,
          rrows0_v, rrows1_v, part_v, red_v, shared_s,
          esem, sem0, sem1):
        cid = lax.axis_index("c")
        sid = lax.axis_index("s")
        wid = sid * 2 + cid
        base = wid * BPW

        pltpu.sync_copy(center_h.at[pl.ds(base, BPW)], cidx_v)
        ecp = pltpu.async_copy(emb_h.at[cidx_v], erows_v, esem)
        pltpu.sync_copy(ctx_h.at[pl.ds(wid * (BPW * C), BPW * C)], xidx_v)
        pltpu.sync_copy(rnd_h.at[pl.ds(wid * (BPW * R), BPW * R)], ridx_v)

        xrows = (xrows0_v, xrows1_v)
        rrows = (rrows0_v, rrows1_v)
        sems = (sem0, sem1)

        def issue(s):
            xbuf, rbuf, sem = xrows[s % 2], rrows[s % 2], sems[s % 2]
            return (
                pltpu.async_copy(lin_h.at[xidx_v.at[pl.ds(s * CC, CC)]],
                                 xbuf, sem),
                pltpu.async_copy(lin_h.at[ridx_v.at[pl.ds(s * RC, RC // 2)]],
                                 rbuf.at[pl.ds(0, RC // 2)], sem),
                pltpu.async_copy(
                    lin_h.at[ridx_v.at[pl.ds(s * RC + RC // 2, RC // 2)]],
                    rbuf.at[pl.ds(RC // 2, RC // 2)], sem),
            )

        lanes = lax.iota(jnp.int32, 16)
        perms = [lanes ^ jnp.int32(1 << p) for p in (3, 2, 1, 0)]
        scale = jnp.where(lanes < C, -1.0 / (B * C),
                          jnp.where(lanes < NCR, -1.0 / (B * R), 0.0))

        cps = issue(0)
        ecp.wait()
        acc = jnp.zeros((16,), jnp.float32)
        for s in range(NSUB):
            nxt = issue(s + 1) if s + 1 < NSUB else None
            for cp in cps:
                cp.wait()
            xbuf, rbuf = xrows[s % 2], rrows[s % 2]

            def b_body(bb, acc_in, s=s, xbuf=xbuf, rbuf=rbuf):
                b = s * SUB + bb
                e = [erows_v[b, pl.ds(16 * t, 16)] for t in range(8)]
                res = jnp.zeros((16,), jnp.float32)
                for j in range(NCR):
                    if j < C:
                        r = bb * C + j
                        wrow = xbuf
                    else:
                        r = bb * R + (j - C)
                        wrow = rbuf
                    d = e[0] * wrow[r, pl.ds(0, 16)]
                    for t in range(1, 8):
                        d = d + e[t] * wrow[r, pl.ds(16 * t, 16)]
                    for p in perms:
                        d = d + jnp.take_along_axis(d, p, axis=0)
                    res = jnp.where(lanes == j, d, res)
                sig = 1.0 / (1.0 + jnp.exp(-res))
                arg = jnp.where(lanes < C, sig, (1.0 + 1e-3) - sig)
                return acc_in + _log_f32(arg) * scale

            acc = lax.fori_loop(0, SUB, b_body, acc)
            cps = nxt

        part_v[...] = acc
        pltpu.sync_copy(part_v, shared_s.at[sid])
        plsc.subcore_barrier()

        @pl.when(sid == 0)
        def _():
            pltpu.sync_copy(shared_s, red_v)
            tot = red_v[0, :]
            for t in range(1, 16):
                tot = tot + red_v[t, :]
            for p in perms:
                tot = tot + jnp.take_along_axis(tot, p, axis=0)
            part_v[...] = tot
            pltpu.sync_copy(part_v, out_h.at[cid])

    return k(center, ctx_idx, rnd_idx, emb_table, lin_w)


def kernel(center, context, rand, emb_table, lin_w):
    part = _sc_loss(
        center.astype(jnp.int32),
        context.astype(jnp.int32).reshape(B * C),
        rand.astype(jnp.int32).reshape(B * R),
        emb_table, lin_w)
    return part[0, 0] + part[1, 0]


# speedup vs baseline: 1.1465x; 1.1465x over previous
import functools

import jax
import jax.numpy as jnp
from jax import lax
from jax.experimental import pallas as pl
from jax.experimental.pallas import tpu as pltpu
from jax.experimental.pallas import tpu_sc as plsc

VOC = 100000
EMB = 128
B = 4096
C = 5
R = 10
NCR = C + R

NW = 32
BPW = B // NW
SUB = 16
NSUB = BPW // SUB
CHIDX = SUB * NCR
HALF = CHIDX // 2
LN2 = 0.6931471805599453


def _log_f32(x):
    bits = lax.bitcast_convert_type(x, jnp.int32)
    e = ((bits >> 23) & 0xFF) - 127
    m = lax.bitcast_convert_type(
        (bits & 0x7FFFFF) | 0x3F800000, jnp.float32)
    s = (m - 1.0) / (m + 1.0)
    s2 = s * s
    p = s * (2.0 + s2 * (2.0 / 3.0 + s2 * (2.0 / 5.0 + s2 * (2.0 / 7.0
             + s2 * (2.0 / 9.0)))))
    return e.astype(jnp.float32) * LN2 + p


def _sc_loss(center, idx_all, emb_table, lin_w):
    mesh = plsc.VectorSubcoreMesh(core_axis_name="c", subcore_axis_name="s")

    @functools.partial(
        pl.kernel,
        mesh=mesh,
        out_type=jax.ShapeDtypeStruct((2, 16), jnp.float32),
        scratch_types=[
            pltpu.VMEM((BPW,), jnp.int32),
            pltpu.VMEM((BPW * NCR,), jnp.int32),
            pltpu.VMEM((BPW, EMB), jnp.float32),
            pltpu.VMEM((CHIDX, EMB), jnp.float32),
            pltpu.VMEM((CHIDX, EMB), jnp.float32),
            pltpu.VMEM((16,), jnp.float32),
            pltpu.VMEM((16, 16), jnp.float32),
            pltpu.VMEM_SHARED((16, 16), jnp.float32),
            pltpu.SemaphoreType.DMA,
            pltpu.SemaphoreType.DMA,
            pltpu.SemaphoreType.DMA,
        ],
    )
    def k(center_h, idx_h, emb_h, lin_h, out_h,
          cidx_v, aidx_v, erows_v, rows0_v, rows1_v,
          part_v, red_v, shared_s, esem, sem0, sem1):
        cid = lax.axis_index("c")
        sid = lax.axis_index("s")
        wid = sid * 2 + cid
        base = wid * BPW

        pltpu.sync_copy(center_h.at[pl.ds(base, BPW)], cidx_v)
        ecp = pltpu.async_copy(emb_h.at[cidx_v], erows_v, esem)
        pltpu.sync_copy(idx_h.at[pl.ds(wid * (BPW * NCR), BPW * NCR)], aidx_v)

        rows = (rows0_v, rows1_v)
        sems = (sem0, sem1)

        def issue(c, par):
            off = pl.multiple_of(c * CHIDX, 8)
            buf, sem = rows[par], sems[par]
            pltpu.async_copy(lin_h.at[aidx_v.at[pl.ds(off, HALF)]],
                             buf.at[pl.ds(0, HALF)], sem)
            pltpu.async_copy(lin_h.at[aidx_v.at[pl.ds(off + HALF, HALF)]],
                             buf.at[pl.ds(HALF, HALF)], sem)

        def wait(par):
            buf, sem = rows[par], sems[par]
            pltpu.make_async_copy(lin_h.at[aidx_v.at[pl.ds(0, HALF)]],
                                  buf.at[pl.ds(0, HALF)], sem).wait()
            pltpu.make_async_copy(lin_h.at[aidx_v.at[pl.ds(0, HALF)]],
                                  buf.at[pl.ds(HALF, HALF)], sem).wait()

        lanes = lax.iota(jnp.int32, 16)
        perms = [lanes ^ jnp.int32(1 << p) for p in (3, 2, 1, 0)]
        scale = jnp.where(lanes < C, -1.0 / (B * C),
                          jnp.where(lanes < NCR, -1.0 / (B * R), 0.0))

        def compute_chunk(c, par, acc_in):
            buf = rows[par]

            def b_body(bb, acc2):
                b = c * SUB + bb
                e = [erows_v[b, pl.ds(16 * t, 16)] for t in range(8)]
                res = jnp.zeros((16,), jnp.float32)
                for j in range(NCR):
                    r = bb * NCR + j
                    d = e[0] * buf[r, pl.ds(0, 16)]
                    for t in range(1, 8):
                        d = d + e[t] * buf[r, pl.ds(16 * t, 16)]
                    for p in perms:
                        d = d + jnp.take_along_axis(d, p, axis=0)
                    res = jnp.where(lanes == j, d, res)
                sig = 1.0 / (1.0 + jnp.exp(-res))
                arg = jnp.where(lanes < C, sig, (1.0 + 1e-3) - sig)
                return acc2 + _log_f32(arg) * scale

            return lax.fori_loop(0, SUB, b_body, acc_in)

        issue(0, 0)
        issue(1, 1)
        ecp.wait()

        def sp_body(sp, acc_in):
            wait(0)
            acc1 = compute_chunk(2 * sp, 0, acc_in)

            @pl.when(sp < NSUB // 2 - 1)
            def _():
                issue(2 * sp + 2, 0)

            wait(1)
            acc2 = compute_chunk(2 * sp + 1, 1, acc1)

            @pl.when(sp < NSUB // 2 - 1)
            def _():
                issue(2 * sp + 3, 1)

            return acc2

        acc = lax.fori_loop(0, NSUB // 2, sp_body,
                            jnp.zeros((16,), jnp.float32))

        part_v[...] = acc
        pltpu.sync_copy(part_v, shared_s.at[sid])
        plsc.subcore_barrier()

        @pl.when(sid == 0)
        def _():
            pltpu.sync_copy(shared_s, red_v)
            tot = red_v[0, :]
            for t in range(1, 16):
                tot = tot + red_v[t, :]
            for p in perms:
                tot = tot + jnp.take_along_axis(tot, p, axis=0)
            part_v[...] = tot
            pltpu.sync_copy(part_v, out_h.at[cid])

    return k(center, idx_all, emb_table, lin_w)


def kernel(center, context, rand, emb_table, lin_w):
    idx_all = jnp.concatenate(
        [context.astype(jnp.int32), rand.astype(jnp.int32)],
        axis=1).reshape(B * NCR)
    part = _sc_loss(center.astype(jnp.int32), idx_all, emb_table, lin_w)
    return part[0, 0] + part[1, 0]


# speedup vs baseline: 1.1560x; 1.0083x over previous
import functools

import jax
import jax.numpy as jnp
from jax import lax
from jax.experimental import pallas as pl
from jax.experimental.pallas import tpu as pltpu
from jax.experimental.pallas import tpu_sc as plsc

VOC = 100000
EMB = 128
B = 4096
C = 5
R = 10
NCR = C + R

NW = 32
BPW = B // NW
SUB = 16
NSUB = BPW // SUB
CHIDX = SUB * NCR
HALF = CHIDX // 2
LN2 = 0.6931471805599453


def _log_f32(x):
    bits = lax.bitcast_convert_type(x, jnp.int32)
    e = ((bits >> 23) & 0xFF) - 127
    m = lax.bitcast_convert_type(
        (bits & 0x7FFFFF) | 0x3F800000, jnp.float32)
    s = (m - 1.0) / (m + 1.0)
    s2 = s * s
    p = s * (2.0 + s2 * (2.0 / 3.0 + s2 * (2.0 / 5.0 + s2 * (2.0 / 7.0
             + s2 * (2.0 / 9.0)))))
    return e.astype(jnp.float32) * LN2 + p


def _sc_loss(center, idx_all, emb_table, lin_w):
    mesh = plsc.VectorSubcoreMesh(core_axis_name="c", subcore_axis_name="s")

    @functools.partial(
        pl.kernel,
        mesh=mesh,
        out_type=jax.ShapeDtypeStruct((2, 16), jnp.float32),
        scratch_types=[
            pltpu.VMEM((BPW,), jnp.int32),
            pltpu.VMEM((BPW * NCR,), jnp.int32),
            pltpu.VMEM((BPW, EMB), jnp.float32),
            pltpu.VMEM((2, CHIDX, EMB), jnp.float32),
            pltpu.VMEM((16,), jnp.float32),
            pltpu.VMEM((16, 16), jnp.float32),
            pltpu.VMEM_SHARED((16, 16), jnp.float32),
            pltpu.SemaphoreType.DMA,
            pltpu.SemaphoreType.DMA,
        ],
    )
    def k(center_h, idx_h, emb_h, lin_h, out_h,
          cidx_v, aidx_v, erows_v, rows_v,
          part_v, red_v, shared_s, esem, sem):
        cid = lax.axis_index("c")
        sid = lax.axis_index("s")
        wid = sid * 2 + cid
        base = wid * BPW

        pltpu.sync_copy(center_h.at[pl.ds(base, BPW)], cidx_v)
        ecp = pltpu.async_copy(emb_h.at[cidx_v], erows_v, esem)
        pltpu.sync_copy(idx_h.at[pl.ds(wid * (BPW * NCR), BPW * NCR)], aidx_v)

        def issue(c):
            off = pl.multiple_of(c * CHIDX, 8)
            buf = rows_v.at[c % 2]
            pltpu.async_copy(lin_h.at[aidx_v.at[pl.ds(off, HALF)]],
                             buf.at[pl.ds(0, HALF)], sem)
            pltpu.async_copy(lin_h.at[aidx_v.at[pl.ds(off + HALF, HALF)]],
                             buf.at[pl.ds(HALF, HALF)], sem)

        def wait_chunk():
            buf = rows_v.at[0]
            pltpu.make_async_copy(lin_h.at[aidx_v.at[pl.ds(0, HALF)]],
                                  buf.at[pl.ds(0, HALF)], sem).wait()
            pltpu.make_async_copy(lin_h.at[aidx_v.at[pl.ds(0, HALF)]],
                                  buf.at[pl.ds(HALF, HALF)], sem).wait()

        lanes = lax.iota(jnp.int32, 16)
        perms = [lanes ^ jnp.int32(1 << p) for p in (3, 2, 1, 0)]
        scale = jnp.where(lanes < C, -1.0 / (B * C),
                          jnp.where(lanes < NCR, -1.0 / (B * R), 0.0))

        issue(0)
        ecp.wait()

        def c_body(c, acc_in):
            wait_chunk()

            @pl.when(c < NSUB - 1)
            def _():
                issue(c + 1)

            par = c % 2

            def b_body(bb, acc2):
                b = c * SUB + bb
                e = [erows_v[b, pl.ds(16 * t, 16)] for t in range(8)]
                res = jnp.zeros((16,), jnp.float32)
                for j in range(NCR):
                    r = bb * NCR + j
                    d = e[0] * rows_v[par, r, pl.ds(0, 16)]
                    for t in range(1, 8):
                        d = d + e[t] * rows_v[par, r, pl.ds(16 * t, 16)]
                    for p in perms:
                        d = d + jnp.take_along_axis(d, p, axis=0)
                    res = jnp.where(lanes == j, d, res)
                sig = 1.0 / (1.0 + jnp.exp(-res))
                arg = jnp.where(lanes < C, sig, (1.0 + 1e-3) - sig)
                return acc2 + _log_f32(arg) * scale

            return lax.fori_loop(0, SUB, b_body, acc_in)

        acc = lax.fori_loop(0, NSUB, c_body, jnp.zeros((16,), jnp.float32))

        part_v[...] = acc
        pltpu.sync_copy(part_v, shared_s.at[sid])
        plsc.subcore_barrier()

        @pl.when(sid == 0)
        def _():
            pltpu.sync_copy(shared_s, red_v)
            tot = red_v[0, :]
            for t in range(1, 16):
                tot = tot + red_v[t, :]
            for p in perms:
                tot = tot + jnp.take_along_axis(tot, p, axis=0)
            part_v[...] = tot
            pltpu.sync_copy(part_v, out_h.at[cid])

    return k(center, idx_all, emb_table, lin_w)


def kernel(center, context, rand, emb_table, lin_w):
    idx_all = jnp.concatenate(
        [context.astype(jnp.int32), rand.astype(jnp.int32)],
        axis=1).reshape(B * NCR)
    part = _sc_loss(center.astype(jnp.int32), idx_all, emb_table, lin_w)
    return part[0, 0] + part[1, 0]
